# baseline (speedup 1.0000x reference)
import functools

import jax
import jax.numpy as jnp
from jax import lax
from jax.experimental import pallas as pl
from jax.experimental.pallas import tpu as pltpu
from jax.experimental.pallas import tpu_sc as plsc

_C = 6625
_D = 512
_B = 64
_S = 80
_NEG = -3.4e38


def _prep_body(label_len_ref, predicts_ref, labels_ref, labs_ref, w_ref):
    b = pl.program_id(0)
    x = predicts_ref[0]
    ci = lax.broadcasted_iota(jnp.int32, (_S, _C), 1)
    xm = jnp.where(ci < _C, x, _NEG)
    m = jnp.max(xm, axis=1, keepdims=True)
    raw = jnp.min(jnp.where(xm == m, ci, _C), axis=1, keepdims=True)

    prev = jnp.concatenate(
        [jnp.full((1, 1), -1, jnp.int32), raw[:-1]], axis=0)
    char_rep = prev == raw
    is_char = raw > 0
    mk = jnp.logical_and(is_char, jnp.logical_not(char_rep)).astype(jnp.float32)

    count = jnp.sum(mk)
    valid = (count == label_len_ref[b].astype(jnp.float32)).astype(jnp.float32)

    ri = lax.broadcasted_iota(jnp.int32, (_S, _S), 0)
    ti = lax.broadcasted_iota(jnp.int32, (_S, _S), 1)
    ltri = (ti <= ri).astype(jnp.float32)
    cs = jnp.dot(ltri, mk, preferred_element_type=jnp.float32)
    rank = jnp.clip(cs.astype(jnp.int32) - 1, 0, _S - 1)

    onehot = (rank == ti).astype(jnp.float32)
    labels_col = labels_ref[0].astype(jnp.float32)
    labs_f = jnp.dot(onehot, labels_col, preferred_element_type=jnp.float32)

    labs_ref[0] = labs_f.astype(jnp.int32)
    w_ref[0] = jnp.broadcast_to(mk * valid, (_S, 16))


def _prep(predicts, labels3, label_len):
    return pl.pallas_call(
        _prep_body,
        grid=(_B,),
        in_specs=[
            pl.BlockSpec(memory_space=pltpu.SMEM),
            pl.BlockSpec((1, _S, _C), lambda b: (b, 0, 0)),
            pl.BlockSpec((1, _S, 1), lambda b: (b, 0, 0)),
        ],
        out_specs=[
            pl.BlockSpec((1, _S, 1), lambda b: (b, 0, 0)),
            pl.BlockSpec((1, _S, 16), lambda b: (b, 0, 0)),
        ],
        out_shape=[
            jax.ShapeDtypeStruct((_B, _S, 1), jnp.int32),
            jax.ShapeDtypeStruct((_B, _S, 16), jnp.float32),
        ],
    )(label_len, predicts, labels3)


_NC, _NS = 2, 16
_NW = _NC * _NS
_N = _B * _S
_PER_W = _N // _NW
_CHUNK = 32
_NCHUNK = _PER_W // _CHUNK


def _sc_body(centers_hbm, labs_hbm, w_hbm, emb_hbm, out_sq_hbm, out_w_hbm,
             idx_v, w_v, c_v, e_v, res_v, sem):
    wid = lax.axis_index("s") * _NC + lax.axis_index("c")
    base = wid * _PER_W
    pltpu.sync_copy(labs_hbm.at[pl.ds(base, _PER_W)], idx_v)
    pltpu.sync_copy(w_hbm.at[pl.ds(base, _PER_W)], w_v)

    acc = jnp.zeros((16,), jnp.float32)
    wacc = jnp.zeros((16,), jnp.float32)
    for g in range(_NCHUNK):
        pltpu.async_copy(
            centers_hbm.at[idx_v.at[pl.ds(g * _CHUNK, _CHUNK)]], c_v, sem
        ).wait()
        pltpu.sync_copy(emb_hbm.at[pl.ds(base + g * _CHUNK, _CHUNK)], e_v)

        def row_body(r, carry):
            acc, wacc = carry
            wspl = w_v[g * _CHUNK + r, :]
            s = jnp.zeros((16,), jnp.float32)
            for k in range(_D // 16):
                ev = e_v[r, pl.ds(k * 16, 16)]
                cv = c_v[r, pl.ds(k * 16, 16)]
                d = ev - cv
                s = s + d * d
            return acc + wspl * s, wacc + wspl

        acc, wacc = lax.fori_loop(0, _CHUNK, row_body, (acc, wacc))

    res_v[0, :] = acc
    res_v[1, :] = wacc
    pltpu.sync_copy(res_v.at[0], out_sq_hbm.at[wid])
    pltpu.sync_copy(res_v.at[1], out_w_hbm.at[wid])


def _sc_loss(centers, labs_flat, w_flat, emb_flat):
    mesh = plsc.VectorSubcoreMesh(
        core_axis_name="c", subcore_axis_name="s")
    run = pl.kernel(
        _sc_body,
        out_type=[
            jax.ShapeDtypeStruct((_NW, 16), jnp.float32),
            jax.ShapeDtypeStruct((_NW, 16), jnp.float32),
        ],
        mesh=mesh,
        scratch_types=[
            pltpu.VMEM((_PER_W,), jnp.int32),
            pltpu.VMEM((_PER_W, 16), jnp.float32),
            pltpu.VMEM((_CHUNK, _D), jnp.float32),
            pltpu.VMEM((_CHUNK, _D), jnp.float32),
            pltpu.VMEM((2, 16), jnp.float32),
            pltpu.SemaphoreType.DMA,
        ],
    )
    return run(centers, labs_flat, w_flat, emb_flat)


@jax.jit
def kernel(predicts, embedding, labels, label_len, centers):
    labels3 = labels.reshape(_B, _S, 1)
    labs, w16 = _prep(predicts, labels3, label_len)
    labs_flat = labs.reshape(_N)
    w16_flat = w16.reshape(_N, 16)
    emb_flat = embedding.reshape(_N, _D)
    part_sq, part_w = _sc_loss(centers, labs_flat, w16_flat, emb_flat)
    total = jnp.sum(part_sq)
    wsum = jnp.sum(part_w) / 16.0
    return total / (wsum * _D)
